# baseline (speedup 1.0000x reference)
import dataclasses
import functools

import jax
import jax.numpy as jnp
from jax import lax
from jax.experimental import pallas as pl
from jax.experimental.pallas import tpu as pltpu
from jax.experimental.pallas import tpu_sc as plsc

_NUM_MARGINALS = 8
_NUM_NODES = 64
_LOWER = -4.0
_UPPER = 4.0
_H = (_UPPER - _LOWER) / (_NUM_NODES - 1)
_INV_H = (_NUM_NODES - 1) / (_UPPER - _LOWER)

_NC = 2
_NS = 16
_NW = _NC * _NS

_PLANE = 224 * 224
_PLANES = 64 * _NUM_MARGINALS
_PLANES_PER_W = _PLANES // _NW
_VECS = _PLANE // 16
_UNROLL = 16
_TAB = 512


def _spline_tables(nodal_values):
    n = _NUM_NODES
    h = _H
    diag = jnp.ones(n, dtype=jnp.float32).at[1:-1].set(4.0)
    sup = jnp.ones(n - 1, dtype=jnp.float32).at[0].set(0.0)
    sub = jnp.ones(n - 1, dtype=jnp.float32).at[-1].set(0.0)
    A = jnp.diag(diag) + jnp.diag(sup, 1) + jnp.diag(sub, -1)
    rhs = jnp.zeros((_NUM_MARGINALS, n), dtype=jnp.float32)
    rhs = rhs.at[:, 1:n - 1].set(
        3.0 * (nodal_values[:, 0:n - 2] - 2.0 * nodal_values[:, 1:n - 1]
               + nodal_values[:, 2:]) / (h ** 2))
    c = jnp.linalg.solve(A, rhs.T).T
    b = (nodal_values[:, 1:] - nodal_values[:, :-1]) / h \
        - h * (2.0 * c[:, :-1] + c[:, 1:]) / 3.0
    d = (c[:, 1:] - c[:, :-1]) / (3.0 * h)
    a_t = nodal_values[:, :-1]
    c_t = c[:, :-1]

    def flat(t):
        return jnp.pad(t, ((0, 0), (0, 1))).reshape(-1)

    return flat(a_t), flat(b), flat(c_t), flat(d)


def _sc_body(x_hbm, a_hbm, b_hbm, c_hbm, d_hbm, out_hbm,
             buf0, buf1, atab, btab, ctab, dtab, accb, sem0, sem1):
    cid = lax.axis_index("core")
    sid = lax.axis_index("subcore")
    wid = sid * _NC + cid
    base_plane = wid * _PLANES_PER_W

    pltpu.sync_copy(a_hbm, atab)
    pltpu.sync_copy(b_hbm, btab)
    pltpu.sync_copy(c_hbm, ctab)
    pltpu.sync_copy(d_hbm, dtab)

    bufs = (buf0, buf1)
    sems = (sem0, sem1)
    pltpu.make_async_copy(x_hbm.at[base_plane], buf0, sem0).start()

    acc = jnp.zeros((16,), jnp.float32)
    for k in range(_PLANES_PER_W):
        buf = bufs[k % 2]
        sem = sems[k % 2]
        if k + 1 < _PLANES_PER_W:
            pltpu.make_async_copy(
                x_hbm.at[base_plane + k + 1], bufs[(k + 1) % 2],
                sems[(k + 1) % 2]).start()
        pltpu.make_async_copy(x_hbm.at[base_plane + k], buf, sem).wait()

        moff = (k % _NUM_MARGINALS) * _NUM_NODES
        asl = atab.at[pl.ds(moff, _NUM_NODES)]
        bsl = btab.at[pl.ds(moff, _NUM_NODES)]
        csl = ctab.at[pl.ds(moff, _NUM_NODES)]
        dsl = dtab.at[pl.ds(moff, _NUM_NODES)]

        def eval_vec(i, buf=buf, asl=asl, bsl=bsl, csl=csl, dsl=dsl):
            xv = buf[pl.ds(i * 16, 16)]
            u = xv * _INV_H + 31.5
            uc = jnp.minimum(jnp.maximum(u, 0.0), float(_NUM_NODES - 2))
            ji = uc.astype(jnp.int32)
            jf = ji.astype(jnp.float32)
            y = (u - jf) * _H
            av = plsc.load_gather(asl, [ji])
            bv = plsc.load_gather(bsl, [ji])
            cv = plsc.load_gather(csl, [ji])
            dv = plsc.load_gather(dsl, [ji])
            return av + y * (bv + y * (cv + y * dv))

        def chunk(it, acc):
            base = it * _UNROLL
            vals = [eval_vec(base + r) for r in range(_UNROLL)]
            while len(vals) > 1:
                vals = [vals[i] + vals[i + 1] for i in range(0, len(vals), 2)]
            return acc + vals[0]

        acc = lax.fori_loop(0, _VECS // _UNROLL, chunk, acc)

    accb[...] = acc
    pltpu.sync_copy(accb, out_hbm.at[wid])


@functools.partial(jax.jit, donate_argnums=())
def kernel(x, nodal_values):
    a_t, b_t, c_t, d_t = _spline_tables(nodal_values)
    x2 = x.reshape(_PLANES, _PLANE)

    cp = pltpu.CompilerParams()
    if "needs_layout_passes" in pltpu.CompilerParams.__dataclass_fields__:
        cp = dataclasses.replace(cp, needs_layout_passes=False)

    mesh = plsc.VectorSubcoreMesh(core_axis_name="core",
                                  subcore_axis_name="subcore")
    partial = pl.kernel(
        _sc_body,
        out_type=jax.ShapeDtypeStruct((_NW, 16), jnp.float32),
        mesh=mesh,
        scratch_types=[
            pltpu.VMEM((_PLANE,), jnp.float32),
            pltpu.VMEM((_PLANE,), jnp.float32),
            pltpu.VMEM((_TAB,), jnp.float32),
            pltpu.VMEM((_TAB,), jnp.float32),
            pltpu.VMEM((_TAB,), jnp.float32),
            pltpu.VMEM((_TAB,), jnp.float32),
            pltpu.VMEM((16,), jnp.float32),
            pltpu.SemaphoreType.DMA,
            pltpu.SemaphoreType.DMA,
        ],
        compiler_params=cp,
    )(x2, a_t, b_t, c_t, d_t)
    return jnp.sum(partial)


# speedup vs baseline: 1.2335x; 1.2335x over previous
import dataclasses
import functools

import jax
import jax.numpy as jnp
from jax import lax
from jax.experimental import pallas as pl
from jax.experimental.pallas import tpu as pltpu
from jax.experimental.pallas import tpu_sc as plsc

_NUM_MARGINALS = 8
_NUM_NODES = 64
_LOWER = -4.0
_UPPER = 4.0
_H = (_UPPER - _LOWER) / (_NUM_NODES - 1)
_INV_H = (_NUM_NODES - 1) / (_UPPER - _LOWER)

_NC = 2
_NS = 16
_NW = _NC * _NS

_PLANE = 224 * 224
_PLANES = 64 * _NUM_MARGINALS
_PLANES_PER_W = _PLANES // _NW
_VECS = _PLANE // 16
_UNROLL = 4
_TAB = 512


def _spline_tables(nodal_values):
    n = _NUM_NODES
    h = _H
    diag = jnp.ones(n, dtype=jnp.float32).at[1:-1].set(4.0)
    sup = jnp.ones(n - 1, dtype=jnp.float32).at[0].set(0.0)
    sub = jnp.ones(n - 1, dtype=jnp.float32).at[-1].set(0.0)
    A = jnp.diag(diag) + jnp.diag(sup, 1) + jnp.diag(sub, -1)
    rhs = jnp.zeros((_NUM_MARGINALS, n), dtype=jnp.float32)
    rhs = rhs.at[:, 1:n - 1].set(
        3.0 * (nodal_values[:, 0:n - 2] - 2.0 * nodal_values[:, 1:n - 1]
               + nodal_values[:, 2:]) / (h ** 2))
    c = jnp.linalg.solve(A, rhs.T).T
    b = (nodal_values[:, 1:] - nodal_values[:, :-1]) / h \
        - h * (2.0 * c[:, :-1] + c[:, 1:]) / 3.0
    d = (c[:, 1:] - c[:, :-1]) / (3.0 * h)
    a_t = nodal_values[:, :-1]
    c_t = c[:, :-1]

    def flat(t):
        return jnp.pad(t, ((0, 0), (0, 1))).reshape(-1)

    return flat(a_t), flat(b), flat(c_t), flat(d)


def _sc_body(x_hbm, a_hbm, b_hbm, c_hbm, d_hbm, out_hbm,
             buf0, buf1, atab, btab, ctab, dtab, accb, sem0, sem1):
    cid = lax.axis_index("core")
    sid = lax.axis_index("subcore")
    wid = sid * _NC + cid
    base_plane = wid * _PLANES_PER_W

    pltpu.sync_copy(a_hbm, atab)
    pltpu.sync_copy(b_hbm, btab)
    pltpu.sync_copy(c_hbm, ctab)
    pltpu.sync_copy(d_hbm, dtab)

    bufs = (buf0, buf1)
    sems = (sem0, sem1)
    pltpu.make_async_copy(x_hbm.at[base_plane], buf0, sem0).start()

    acc = jnp.zeros((16,), jnp.float32)
    for k in range(_PLANES_PER_W):
        buf = bufs[k % 2]
        sem = sems[k % 2]
        if k + 1 < _PLANES_PER_W:
            pltpu.make_async_copy(
                x_hbm.at[base_plane + k + 1], bufs[(k + 1) % 2],
                sems[(k + 1) % 2]).start()
        pltpu.make_async_copy(x_hbm.at[base_plane + k], buf, sem).wait()

        moff = (k % _NUM_MARGINALS) * _NUM_NODES
        asl = atab.at[pl.ds(moff, _NUM_NODES)]
        bsl = btab.at[pl.ds(moff, _NUM_NODES)]
        csl = ctab.at[pl.ds(moff, _NUM_NODES)]
        dsl = dtab.at[pl.ds(moff, _NUM_NODES)]

        def eval_vec(i, acc, buf=buf, asl=asl, bsl=bsl, csl=csl, dsl=dsl):
            xv = buf[pl.ds(i * 16, 16)]
            u = xv * _INV_H + 31.5
            uc = jnp.minimum(jnp.maximum(u, 0.0), float(_NUM_NODES - 2))
            ji = uc.astype(jnp.int32)
            jf = ji.astype(jnp.float32)
            y = (u - jf) * _H
            av = plsc.load_gather(asl, [ji])
            bv = plsc.load_gather(bsl, [ji])
            cv = plsc.load_gather(csl, [ji])
            dv = plsc.load_gather(dsl, [ji])
            return acc + (av + y * (bv + y * (cv + y * dv)))

        acc = plsc.parallel_loop(0, _VECS, 1, unroll=_UNROLL,
                                 carry=acc)(eval_vec)

    accb[...] = acc
    pltpu.sync_copy(accb, out_hbm.at[wid])


@functools.partial(jax.jit, donate_argnums=())
def kernel(x, nodal_values):
    a_t, b_t, c_t, d_t = _spline_tables(nodal_values)
    x2 = x.reshape(_PLANES, _PLANE)

    cp = pltpu.CompilerParams()
    if "needs_layout_passes" in pltpu.CompilerParams.__dataclass_fields__:
        cp = dataclasses.replace(cp, needs_layout_passes=False)

    mesh = plsc.VectorSubcoreMesh(core_axis_name="core",
                                  subcore_axis_name="subcore")
    partial = pl.kernel(
        _sc_body,
        out_type=jax.ShapeDtypeStruct((_NW, 16), jnp.float32),
        mesh=mesh,
        scratch_types=[
            pltpu.VMEM((_PLANE,), jnp.float32),
            pltpu.VMEM((_PLANE,), jnp.float32),
            pltpu.VMEM((_TAB,), jnp.float32),
            pltpu.VMEM((_TAB,), jnp.float32),
            pltpu.VMEM((_TAB,), jnp.float32),
            pltpu.VMEM((_TAB,), jnp.float32),
            pltpu.VMEM((16,), jnp.float32),
            pltpu.SemaphoreType.DMA,
            pltpu.SemaphoreType.DMA,
        ],
        compiler_params=cp,
    )(x2, a_t, b_t, c_t, d_t)
    return jnp.sum(partial)


# speedup vs baseline: 1.3385x; 1.0851x over previous
import dataclasses
import functools

import numpy as np

import jax
import jax.numpy as jnp
from jax import lax
from jax.experimental import pallas as pl
from jax.experimental.pallas import tpu as pltpu
from jax.experimental.pallas import tpu_sc as plsc

_NUM_MARGINALS = 8
_NUM_NODES = 64
_LOWER = -4.0
_UPPER = 4.0
_H = (_UPPER - _LOWER) / (_NUM_NODES - 1)
_INV_H = (_NUM_NODES - 1) / (_UPPER - _LOWER)

_NC = 2
_NS = 16
_NW = _NC * _NS

_PLANE = 224 * 224
_PLANES = 64 * _NUM_MARGINALS
_PLANES_PER_W = _PLANES // _NW
_VECS = _PLANE // 16
_UNROLL = 4
_TAB = 512


def _tridiag_inv() -> np.ndarray:
    n = _NUM_NODES
    A = np.diag(np.r_[1.0, np.full(n - 2, 4.0), 1.0])
    A += np.diag(np.r_[0.0, np.ones(n - 2)], 1)
    A += np.diag(np.r_[np.ones(n - 2), 0.0], -1)
    return np.linalg.inv(A).astype(np.float32)


_A_INV_T = _tridiag_inv().T


def _spline_tables(nodal_values):
    n = _NUM_NODES
    h = _H
    rhs = jnp.zeros((_NUM_MARGINALS, n), dtype=jnp.float32)
    rhs = rhs.at[:, 1:n - 1].set(
        3.0 * (nodal_values[:, 0:n - 2] - 2.0 * nodal_values[:, 1:n - 1]
               + nodal_values[:, 2:]) / (h ** 2))
    c = rhs @ jnp.asarray(_A_INV_T)
    b = (nodal_values[:, 1:] - nodal_values[:, :-1]) / h \
        - h * (2.0 * c[:, :-1] + c[:, 1:]) / 3.0
    d = (c[:, 1:] - c[:, :-1]) / (3.0 * h)
    a_t = nodal_values[:, :-1]
    c_t = c[:, :-1]

    def flat(t):
        return jnp.pad(t, ((0, 0), (0, 1))).reshape(-1)

    return flat(a_t), flat(b), flat(c_t), flat(d)


def _sc_body(x_hbm, a_hbm, b_hbm, c_hbm, d_hbm, out_hbm,
             buf0, buf1, atab, btab, ctab, dtab, accb, sem0, sem1):
    cid = lax.axis_index("core")
    sid = lax.axis_index("subcore")
    wid = sid * _NC + cid
    base_plane = wid * _PLANES_PER_W

    pltpu.sync_copy(a_hbm, atab)
    pltpu.sync_copy(b_hbm, btab)
    pltpu.sync_copy(c_hbm, ctab)
    pltpu.sync_copy(d_hbm, dtab)

    bufs = (buf0, buf1)
    sems = (sem0, sem1)
    pltpu.make_async_copy(x_hbm.at[base_plane], buf0, sem0).start()

    acc = jnp.zeros((16,), jnp.float32)
    for k in range(_PLANES_PER_W):
        buf = bufs[k % 2]
        sem = sems[k % 2]
        if k + 1 < _PLANES_PER_W:
            pltpu.make_async_copy(
                x_hbm.at[base_plane + k + 1], bufs[(k + 1) % 2],
                sems[(k + 1) % 2]).start()
        pltpu.make_async_copy(x_hbm.at[base_plane + k], buf, sem).wait()

        moff = (k % _NUM_MARGINALS) * _NUM_NODES
        asl = atab.at[pl.ds(moff, _NUM_NODES)]
        bsl = btab.at[pl.ds(moff, _NUM_NODES)]
        csl = ctab.at[pl.ds(moff, _NUM_NODES)]
        dsl = dtab.at[pl.ds(moff, _NUM_NODES)]

        def eval_vec(i, acc, buf=buf, asl=asl, bsl=bsl, csl=csl, dsl=dsl):
            xv = buf[pl.ds(i * 16, 16)]
            u = xv * _INV_H + 31.5
            uc = jnp.minimum(jnp.maximum(u, 0.0), float(_NUM_NODES - 2))
            ji = uc.astype(jnp.int32)
            jf = ji.astype(jnp.float32)
            y = (u - jf) * _H
            av = plsc.load_gather(asl, [ji])
            bv = plsc.load_gather(bsl, [ji])
            cv = plsc.load_gather(csl, [ji])
            dv = plsc.load_gather(dsl, [ji])
            return acc + (av + y * (bv + y * (cv + y * dv)))

        acc = plsc.parallel_loop(0, _VECS, 1, unroll=_UNROLL,
                                 carry=acc)(eval_vec)

    accb[...] = acc
    pltpu.sync_copy(accb, out_hbm.at[wid])


@functools.partial(jax.jit, donate_argnums=())
def kernel(x, nodal_values):
    a_t, b_t, c_t, d_t = _spline_tables(nodal_values)
    x2 = x.reshape(_PLANES, _PLANE)

    cp = pltpu.CompilerParams()
    if "needs_layout_passes" in pltpu.CompilerParams.__dataclass_fields__:
        cp = dataclasses.replace(cp, needs_layout_passes=False)

    mesh = plsc.VectorSubcoreMesh(core_axis_name="core",
                                  subcore_axis_name="subcore")
    partial = pl.kernel(
        _sc_body,
        out_type=jax.ShapeDtypeStruct((_NW, 16), jnp.float32),
        mesh=mesh,
        scratch_types=[
            pltpu.VMEM((_PLANE,), jnp.float32),
            pltpu.VMEM((_PLANE,), jnp.float32),
            pltpu.VMEM((_TAB,), jnp.float32),
            pltpu.VMEM((_TAB,), jnp.float32),
            pltpu.VMEM((_TAB,), jnp.float32),
            pltpu.VMEM((_TAB,), jnp.float32),
            pltpu.VMEM((16,), jnp.float32),
            pltpu.SemaphoreType.DMA,
            pltpu.SemaphoreType.DMA,
        ],
        compiler_params=cp,
    )(x2, a_t, b_t, c_t, d_t)
    return jnp.sum(partial)


# speedup vs baseline: 1.6667x; 1.2452x over previous
import dataclasses
import functools

import numpy as np

import jax
import jax.numpy as jnp
from jax import lax
from jax.experimental import pallas as pl
from jax.experimental.pallas import tpu as pltpu
from jax.experimental.pallas import tpu_sc as plsc

_NUM_MARGINALS = 8
_NUM_NODES = 64
_LOWER = -4.0
_UPPER = 4.0
_H = (_UPPER - _LOWER) / (_NUM_NODES - 1)
_INV_H = (_NUM_NODES - 1) / (_UPPER - _LOWER)

_NC = 2
_NS = 16
_NW = _NC * _NS

_PLANE = 224 * 224
_PLANES = 64 * _NUM_MARGINALS
_PLANES_PER_W = _PLANES // _NW
_VECS = _PLANE // 16
_UNROLL = 4
_TAB = 512


def _tridiag_inv() -> np.ndarray:
    n = _NUM_NODES
    A = np.diag(np.r_[1.0, np.full(n - 2, 4.0), 1.0])
    A += np.diag(np.r_[0.0, np.ones(n - 2)], 1)
    A += np.diag(np.r_[np.ones(n - 2), 0.0], -1)
    return np.linalg.inv(A).astype(np.float32)


_A_INV_T = _tridiag_inv().T


def _spline_tables(nodal_values):
    n = _NUM_NODES
    h = _H
    rhs = jnp.zeros((_NUM_MARGINALS, n), dtype=jnp.float32)
    rhs = rhs.at[:, 1:n - 1].set(
        3.0 * (nodal_values[:, 0:n - 2] - 2.0 * nodal_values[:, 1:n - 1]
               + nodal_values[:, 2:]) / (h ** 2))
    c = rhs @ jnp.asarray(_A_INV_T)
    b = (nodal_values[:, 1:] - nodal_values[:, :-1]) / h \
        - h * (2.0 * c[:, :-1] + c[:, 1:]) / 3.0
    d = (c[:, 1:] - c[:, :-1]) / (3.0 * h)
    a_t = nodal_values[:, :-1]
    c_t = c[:, :-1]

    def flat(t):
        return jnp.pad(t, ((0, 0), (0, 1))).reshape(-1)

    return flat(a_t), flat(b), flat(c_t), flat(d)


_ROWS = 224
_RVECS = 224 // 16
_MAGIC = 4682


def _sc_body(x_hbm, a_hbm, b_hbm, c_hbm, d_hbm, out_hbm,
             buf0, buf1, atab, btab, ctab, dtab, accb, sem0, sem1):
    cid = lax.axis_index("core")
    sid = lax.axis_index("subcore")
    wid = sid * _NC + cid
    base_plane = wid * _PLANES_PER_W

    pltpu.sync_copy(a_hbm, atab)
    pltpu.sync_copy(b_hbm, btab)
    pltpu.sync_copy(c_hbm, ctab)
    pltpu.sync_copy(d_hbm, dtab)

    bufs = (buf0, buf1)
    sems = (sem0, sem1)
    pltpu.make_async_copy(x_hbm.at[base_plane], buf0, sem0).start()

    acc = jnp.zeros((16,), jnp.float32)
    for k in range(_PLANES_PER_W):
        buf = bufs[k % 2]
        sem = sems[k % 2]
        if k + 1 < _PLANES_PER_W:
            pltpu.make_async_copy(
                x_hbm.at[base_plane + k + 1], bufs[(k + 1) % 2],
                sems[(k + 1) % 2]).start()
        pltpu.make_async_copy(x_hbm.at[base_plane + k], buf, sem).wait()

        moff = (k % _NUM_MARGINALS) * _NUM_NODES
        asl = atab.at[pl.ds(moff, _NUM_NODES)]
        bsl = btab.at[pl.ds(moff, _NUM_NODES)]
        csl = ctab.at[pl.ds(moff, _NUM_NODES)]
        dsl = dtab.at[pl.ds(moff, _NUM_NODES)]

        def eval_vec(i, acc, buf=buf, asl=asl, bsl=bsl, csl=csl, dsl=dsl):
            r = lax.shift_right_logical(i * _MAGIC, 16)
            col = (i - r * _RVECS) * 16
            xv = buf[r, pl.ds(col, 16)]
            u = xv * _INV_H + 31.5
            uc = jnp.minimum(jnp.maximum(u, 0.0), float(_NUM_NODES - 2))
            ji = uc.astype(jnp.int32)
            jf = ji.astype(jnp.float32)
            y = (u - jf) * _H
            av = plsc.load_gather(asl, [ji])
            bv = plsc.load_gather(bsl, [ji])
            cv = plsc.load_gather(csl, [ji])
            dv = plsc.load_gather(dsl, [ji])
            return acc + (av + y * (bv + y * (cv + y * dv)))

        acc = plsc.parallel_loop(0, _VECS, 1, unroll=_UNROLL,
                                 carry=acc)(eval_vec)

    accb[...] = acc
    pltpu.sync_copy(accb, out_hbm.at[pl.ds(wid * 16, 16)])


@functools.partial(jax.jit, donate_argnums=())
def kernel(x, nodal_values):
    a_t, b_t, c_t, d_t = _spline_tables(nodal_values)
    x2 = x.reshape(_PLANES, _ROWS, _ROWS)

    cp = pltpu.CompilerParams(use_tc_tiling_on_sc=True)
    if "needs_layout_passes" in pltpu.CompilerParams.__dataclass_fields__:
        cp = dataclasses.replace(cp, needs_layout_passes=False)

    mesh = plsc.VectorSubcoreMesh(core_axis_name="core",
                                  subcore_axis_name="subcore")
    partial = pl.kernel(
        _sc_body,
        out_type=jax.ShapeDtypeStruct((_NW * 16,), jnp.float32),
        mesh=mesh,
        scratch_types=[
            pltpu.VMEM((_ROWS, _ROWS), jnp.float32),
            pltpu.VMEM((_ROWS, _ROWS), jnp.float32),
            pltpu.VMEM((_TAB,), jnp.float32),
            pltpu.VMEM((_TAB,), jnp.float32),
            pltpu.VMEM((_TAB,), jnp.float32),
            pltpu.VMEM((_TAB,), jnp.float32),
            pltpu.VMEM((16,), jnp.float32),
            pltpu.SemaphoreType.DMA,
            pltpu.SemaphoreType.DMA,
        ],
        compiler_params=cp,
    )(x2, a_t, b_t, c_t, d_t)
    return jnp.sum(partial)


# speedup vs baseline: 1.9788x; 1.1873x over previous
import dataclasses
import functools

import numpy as np

import jax
import jax.numpy as jnp
from jax import lax
from jax.experimental import pallas as pl
from jax.experimental.pallas import tpu as pltpu
from jax.experimental.pallas import tpu_sc as plsc

_NUM_MARGINALS = 8
_NUM_NODES = 64
_LOWER = -4.0
_UPPER = 4.0
_H = (_UPPER - _LOWER) / (_NUM_NODES - 1)
_INV_H = (_NUM_NODES - 1) / (_UPPER - _LOWER)

_NC = 2
_NS = 16
_NW = _NC * _NS

_PLANE = 224 * 224
_PLANES = 64 * _NUM_MARGINALS
_PLANES_PER_W = _PLANES // _NW
_VECS = _PLANE // 16
_UNROLL = 4
_TAB = 512


def _tridiag_inv() -> np.ndarray:
    n = _NUM_NODES
    A = np.diag(np.r_[1.0, np.full(n - 2, 4.0), 1.0])
    A += np.diag(np.r_[0.0, np.ones(n - 2)], 1)
    A += np.diag(np.r_[np.ones(n - 2), 0.0], -1)
    return np.linalg.inv(A).astype(np.float32)


_A_INV_T = _tridiag_inv().T


def _spline_tables(nodal_values):
    n = _NUM_NODES
    h = _H
    rhs = jnp.zeros((_NUM_MARGINALS, n), dtype=jnp.float32)
    rhs = rhs.at[:, 1:n - 1].set(
        3.0 * (nodal_values[:, 0:n - 2] - 2.0 * nodal_values[:, 1:n - 1]
               + nodal_values[:, 2:]) / (h ** 2))
    c = rhs @ jnp.asarray(_A_INV_T)
    b = (nodal_values[:, 1:] - nodal_values[:, :-1]) / h \
        - h * (2.0 * c[:, :-1] + c[:, 1:]) / 3.0
    d = (c[:, 1:] - c[:, :-1]) / (3.0 * h)
    a_t = nodal_values[:, :-1]
    c_t = c[:, :-1]

    def flat(t):
        return jnp.pad(t, ((0, 0), (0, 1))).reshape(-1)

    return flat(a_t), flat(b), flat(c_t), flat(d)


_ROWS = 224
_RVECS = 224 // 16
_MAGIC = 4682


def _sc_body(x_hbm, a_hbm, b_hbm, c_hbm, d_hbm, out_hbm,
             buf0, buf1, atab, btab, ctab, dtab, accb, sem0, sem1):
    cid = lax.axis_index("core")
    sid = lax.axis_index("subcore")
    wid = sid * _NC + cid
    base_plane = wid * _PLANES_PER_W

    pltpu.sync_copy(a_hbm, atab)
    pltpu.sync_copy(b_hbm, btab)
    pltpu.sync_copy(c_hbm, ctab)
    pltpu.sync_copy(d_hbm, dtab)

    bufs = (buf0, buf1)
    sems = (sem0, sem1)
    pltpu.make_async_copy(x_hbm.at[base_plane], buf0, sem0).start()

    acc = jnp.zeros((16,), jnp.float32)
    for k in range(_PLANES_PER_W):
        buf = bufs[k % 2]
        sem = sems[k % 2]
        if k + 1 < _PLANES_PER_W:
            pltpu.make_async_copy(
                x_hbm.at[base_plane + k + 1], bufs[(k + 1) % 2],
                sems[(k + 1) % 2]).start()
        pltpu.make_async_copy(x_hbm.at[base_plane + k], buf, sem).wait()

        moff = (k % _NUM_MARGINALS) * _NUM_NODES
        asl = atab.at[pl.ds(moff, _NUM_NODES)]
        bsl = btab.at[pl.ds(moff, _NUM_NODES)]
        csl = ctab.at[pl.ds(moff, _NUM_NODES)]
        dsl = dtab.at[pl.ds(moff, _NUM_NODES)]

        def eval_vec(xv, asl=asl, bsl=bsl, csl=csl, dsl=dsl):
            u = xv * _INV_H + 31.5
            uc = jnp.minimum(jnp.maximum(u, 0.0), float(_NUM_NODES - 2))
            ji = uc.astype(jnp.int32)
            jf = ji.astype(jnp.float32)
            y = (u - jf) * _H
            av = plsc.load_gather(asl, [ji])
            bv = plsc.load_gather(bsl, [ji])
            cv = plsc.load_gather(csl, [ji])
            dv = plsc.load_gather(dsl, [ji])
            return av + y * (bv + y * (cv + y * dv))

        def tree(vals):
            while len(vals) > 1:
                vals = ([vals[j] + vals[j + 1] for j in range(0, len(vals) - 1, 2)]
                        + ([vals[-1]] if len(vals) % 2 else []))
            return vals[0]

        def eval_half(cbase):
            def body(r, acc, buf=buf, cbase=cbase):
                return acc + tree([eval_vec(buf[r, pl.ds(cbase + k * 16, 16)])
                                   for k in range(_RVECS // 2)])
            return body

        acc = plsc.parallel_loop(0, _ROWS, 1, unroll=1,
                                 carry=acc)(eval_half(0))
        acc = plsc.parallel_loop(0, _ROWS, 1, unroll=1,
                                 carry=acc)(eval_half(_RVECS // 2 * 16))

    accb[...] = acc
    pltpu.sync_copy(accb, out_hbm.at[pl.ds(wid * 16, 16)])


@functools.partial(jax.jit, donate_argnums=())
def kernel(x, nodal_values):
    a_t, b_t, c_t, d_t = _spline_tables(nodal_values)
    x2 = x.reshape(_PLANES, _ROWS, _ROWS)

    cp = pltpu.CompilerParams(use_tc_tiling_on_sc=True)
    if "needs_layout_passes" in pltpu.CompilerParams.__dataclass_fields__:
        cp = dataclasses.replace(cp, needs_layout_passes=False)

    mesh = plsc.VectorSubcoreMesh(core_axis_name="core",
                                  subcore_axis_name="subcore")
    partial = pl.kernel(
        _sc_body,
        out_type=jax.ShapeDtypeStruct((_NW * 16,), jnp.float32),
        mesh=mesh,
        scratch_types=[
            pltpu.VMEM((_ROWS, _ROWS), jnp.float32),
            pltpu.VMEM((_ROWS, _ROWS), jnp.float32),
            pltpu.VMEM((_TAB,), jnp.float32),
            pltpu.VMEM((_TAB,), jnp.float32),
            pltpu.VMEM((_TAB,), jnp.float32),
            pltpu.VMEM((_TAB,), jnp.float32),
            pltpu.VMEM((16,), jnp.float32),
            pltpu.SemaphoreType.DMA,
            pltpu.SemaphoreType.DMA,
        ],
        compiler_params=cp,
    )(x2, a_t, b_t, c_t, d_t)
    return jnp.sum(partial)


# speedup vs baseline: 1.9796x; 1.0004x over previous
import dataclasses
import functools

import numpy as np

import jax
import jax.numpy as jnp
from jax import lax
from jax.experimental import pallas as pl
from jax.experimental.pallas import tpu as pltpu
from jax.experimental.pallas import tpu_sc as plsc

_NUM_MARGINALS = 8
_NUM_NODES = 64
_LOWER = -4.0
_UPPER = 4.0
_H = (_UPPER - _LOWER) / (_NUM_NODES - 1)
_INV_H = (_NUM_NODES - 1) / (_UPPER - _LOWER)

_NC = 2
_NS = 16
_NW = _NC * _NS

_PLANE = 224 * 224
_PLANES = 64 * _NUM_MARGINALS
_PLANES_PER_W = _PLANES // _NW
_VECS = _PLANE // 16
_UNROLL = 4
_TAB = 512


def _tridiag_inv() -> np.ndarray:
    n = _NUM_NODES
    A = np.diag(np.r_[1.0, np.full(n - 2, 4.0), 1.0])
    A += np.diag(np.r_[0.0, np.ones(n - 2)], 1)
    A += np.diag(np.r_[np.ones(n - 2), 0.0], -1)
    return np.linalg.inv(A).astype(np.float32)


_A_INV_T = _tridiag_inv().T


def _spline_tables(nodal_values):
    n = _NUM_NODES
    h = _H
    rhs = jnp.zeros((_NUM_MARGINALS, n), dtype=jnp.float32)
    rhs = rhs.at[:, 1:n - 1].set(
        3.0 * (nodal_values[:, 0:n - 2] - 2.0 * nodal_values[:, 1:n - 1]
               + nodal_values[:, 2:]) / (h ** 2))
    c = jnp.dot(rhs, jnp.asarray(_A_INV_T),
                precision=jax.lax.Precision.HIGHEST)
    b = (nodal_values[:, 1:] - nodal_values[:, :-1]) / h \
        - h * (2.0 * c[:, :-1] + c[:, 1:]) / 3.0
    d = (c[:, 1:] - c[:, :-1]) / (3.0 * h)
    a_t = nodal_values[:, :-1]
    c_t = c[:, :-1]

    def flat(t):
        return jnp.pad(t, ((0, 0), (0, 1))).reshape(-1)

    return flat(a_t), flat(b), flat(c_t), flat(d)


_ROWS = 224
_RVECS = 224 // 16
_MAGIC = 4682


def _sc_body(x_hbm, a_hbm, b_hbm, c_hbm, d_hbm, out_hbm,
             buf0, buf1, atab, btab, ctab, dtab, accb, sem0, sem1):
    cid = lax.axis_index("core")
    sid = lax.axis_index("subcore")
    wid = sid * _NC + cid
    base_plane = wid * _PLANES_PER_W

    pltpu.sync_copy(a_hbm, atab)
    pltpu.sync_copy(b_hbm, btab)
    pltpu.sync_copy(c_hbm, ctab)
    pltpu.sync_copy(d_hbm, dtab)

    bufs = (buf0, buf1)
    sems = (sem0, sem1)
    pltpu.make_async_copy(x_hbm.at[base_plane], buf0, sem0).start()

    acc = jnp.zeros((16,), jnp.float32)
    for k in range(_PLANES_PER_W):
        buf = bufs[k % 2]
        sem = sems[k % 2]
        if k + 1 < _PLANES_PER_W:
            pltpu.make_async_copy(
                x_hbm.at[base_plane + k + 1], bufs[(k + 1) % 2],
                sems[(k + 1) % 2]).start()
        pltpu.make_async_copy(x_hbm.at[base_plane + k], buf, sem).wait()

        moff = (k % _NUM_MARGINALS) * _NUM_NODES
        asl = atab.at[pl.ds(moff, _NUM_NODES)]
        bsl = btab.at[pl.ds(moff, _NUM_NODES)]
        csl = ctab.at[pl.ds(moff, _NUM_NODES)]
        dsl = dtab.at[pl.ds(moff, _NUM_NODES)]

        def eval_vec(xv, asl=asl, bsl=bsl, csl=csl, dsl=dsl):
            u = xv * _INV_H + 31.5
            uc = jnp.minimum(jnp.maximum(u, 0.0), float(_NUM_NODES - 2))
            ji = uc.astype(jnp.int32)
            jf = ji.astype(jnp.float32)
            y = (u - jf) * _H
            av = plsc.load_gather(asl, [ji])
            bv = plsc.load_gather(bsl, [ji])
            cv = plsc.load_gather(csl, [ji])
            dv = plsc.load_gather(dsl, [ji])
            return av + y * (bv + y * (cv + y * dv))

        def tree(vals):
            while len(vals) > 1:
                vals = ([vals[j] + vals[j + 1] for j in range(0, len(vals) - 1, 2)]
                        + ([vals[-1]] if len(vals) % 2 else []))
            return vals[0]

        def eval_half(cbase):
            def body(r, acc, buf=buf, cbase=cbase):
                return acc + tree([eval_vec(buf[r, pl.ds(cbase + k * 16, 16)])
                                   for k in range(_RVECS // 2)])
            return body

        acc = plsc.parallel_loop(0, _ROWS, 1, unroll=1,
                                 carry=acc)(eval_half(0))
        acc = plsc.parallel_loop(0, _ROWS, 1, unroll=1,
                                 carry=acc)(eval_half(_RVECS // 2 * 16))

    accb[...] = acc
    pltpu.sync_copy(accb, out_hbm.at[pl.ds(wid * 16, 16)])


@functools.partial(jax.jit, donate_argnums=())
def kernel(x, nodal_values):
    a_t, b_t, c_t, d_t = _spline_tables(nodal_values)
    x2 = x.reshape(_PLANES, _ROWS, _ROWS)

    cp = pltpu.CompilerParams(use_tc_tiling_on_sc=True)
    if "needs_layout_passes" in pltpu.CompilerParams.__dataclass_fields__:
        cp = dataclasses.replace(cp, needs_layout_passes=False)

    mesh = plsc.VectorSubcoreMesh(core_axis_name="core",
                                  subcore_axis_name="subcore")
    partial = pl.kernel(
        _sc_body,
        out_type=jax.ShapeDtypeStruct((_NW * 16,), jnp.float32),
        mesh=mesh,
        scratch_types=[
            pltpu.VMEM((_ROWS, _ROWS), jnp.float32),
            pltpu.VMEM((_ROWS, _ROWS), jnp.float32),
            pltpu.VMEM((_TAB,), jnp.float32),
            pltpu.VMEM((_TAB,), jnp.float32),
            pltpu.VMEM((_TAB,), jnp.float32),
            pltpu.VMEM((_TAB,), jnp.float32),
            pltpu.VMEM((16,), jnp.float32),
            pltpu.SemaphoreType.DMA,
            pltpu.SemaphoreType.DMA,
        ],
        compiler_params=cp,
    )(x2, a_t, b_t, c_t, d_t)
    return jnp.sum(partial)


# speedup vs baseline: 1.9800x; 1.0002x over previous
import dataclasses
import functools

import numpy as np

import jax
import jax.numpy as jnp
from jax import lax
from jax.experimental import pallas as pl
from jax.experimental.pallas import tpu as pltpu
from jax.experimental.pallas import tpu_sc as plsc

_NUM_MARGINALS = 8
_NUM_NODES = 64
_LOWER = -4.0
_UPPER = 4.0
_H = (_UPPER - _LOWER) / (_NUM_NODES - 1)
_INV_H = (_NUM_NODES - 1) / (_UPPER - _LOWER)

_NC = 2
_NS = 16
_NW = _NC * _NS

_PLANE = 224 * 224
_PLANES = 64 * _NUM_MARGINALS
_PLANES_PER_W = _PLANES // _NW
_TAB = 512


def _tridiag_inv() -> np.ndarray:
    n = _NUM_NODES
    A = np.diag(np.r_[1.0, np.full(n - 2, 4.0), 1.0])
    A += np.diag(np.r_[0.0, np.ones(n - 2)], 1)
    A += np.diag(np.r_[np.ones(n - 2), 0.0], -1)
    return np.linalg.inv(A).astype(np.float32)


_A_INV_T = _tridiag_inv().T


def _spline_tables(nodal_values):
    n = _NUM_NODES
    h = _H
    rhs = jnp.zeros((_NUM_MARGINALS, n), dtype=jnp.float32)
    rhs = rhs.at[:, 1:n - 1].set(
        3.0 * (nodal_values[:, 0:n - 2] - 2.0 * nodal_values[:, 1:n - 1]
               + nodal_values[:, 2:]) / (h ** 2))
    c = jnp.dot(rhs, jnp.asarray(_A_INV_T),
                precision=jax.lax.Precision.HIGHEST)
    b = (nodal_values[:, 1:] - nodal_values[:, :-1]) / h \
        - h * (2.0 * c[:, :-1] + c[:, 1:]) / 3.0
    d = (c[:, 1:] - c[:, :-1]) / (3.0 * h)
    a_t = nodal_values[:, :-1]
    c_t = c[:, :-1]

    def flat(t):
        return jnp.pad(t, ((0, 0), (0, 1))).reshape(-1)

    return flat(a_t), flat(b), flat(c_t), flat(d)


_ROWS = 224
_RVECS = 224 // 16


def _sc_body(x_hbm, a_hbm, b_hbm, c_hbm, d_hbm, out_hbm,
             buf0, buf1, atab, btab, ctab, dtab, accb, sem0, sem1):
    cid = lax.axis_index("core")
    sid = lax.axis_index("subcore")
    wid = sid * _NC + cid
    base_plane = wid * _PLANES_PER_W

    pltpu.sync_copy(a_hbm, atab)
    pltpu.sync_copy(b_hbm, btab)
    pltpu.sync_copy(c_hbm, ctab)
    pltpu.sync_copy(d_hbm, dtab)

    bufs = (buf0, buf1)
    sems = (sem0, sem1)
    pltpu.make_async_copy(x_hbm.at[base_plane], buf0, sem0).start()

    acc = jnp.zeros((16,), jnp.float32)
    for k in range(_PLANES_PER_W):
        buf = bufs[k % 2]
        sem = sems[k % 2]
        if k + 1 < _PLANES_PER_W:
            pltpu.make_async_copy(
                x_hbm.at[base_plane + k + 1], bufs[(k + 1) % 2],
                sems[(k + 1) % 2]).start()
        pltpu.make_async_copy(x_hbm.at[base_plane + k], buf, sem).wait()

        moff = (k % _NUM_MARGINALS) * _NUM_NODES
        asl = atab.at[pl.ds(moff, _NUM_NODES)]
        bsl = btab.at[pl.ds(moff, _NUM_NODES)]
        csl = ctab.at[pl.ds(moff, _NUM_NODES)]
        dsl = dtab.at[pl.ds(moff, _NUM_NODES)]

        def eval_vec(xv, asl=asl, bsl=bsl, csl=csl, dsl=dsl):
            u = xv * _INV_H + 31.5
            uc = jnp.minimum(jnp.maximum(u, 0.0), float(_NUM_NODES - 2))
            ji = uc.astype(jnp.int32)
            jf = ji.astype(jnp.float32)
            y = (u - jf) * _H
            av = plsc.load_gather(asl, [ji])
            bv = plsc.load_gather(bsl, [ji])
            cv = plsc.load_gather(csl, [ji])
            dv = plsc.load_gather(dsl, [ji])
            return av + y * (bv + y * (cv + y * dv))

        def tree(vals):
            while len(vals) > 1:
                vals = ([vals[j] + vals[j + 1] for j in range(0, len(vals) - 1, 2)]
                        + ([vals[-1]] if len(vals) % 2 else []))
            return vals[0]

        def eval_half(cbase):
            def body(r, acc, buf=buf, cbase=cbase):
                return acc + tree([eval_vec(buf[r, pl.ds(cbase + k * 16, 16)])
                                   for k in range(_RVECS // 2)])
            return body

        acc = plsc.parallel_loop(0, _ROWS, 1, unroll=1,
                                 carry=acc)(eval_half(0))
        acc = plsc.parallel_loop(0, _ROWS, 1, unroll=1,
                                 carry=acc)(eval_half(_RVECS // 2 * 16))

    accb[...] = acc
    pltpu.sync_copy(accb, out_hbm.at[pl.ds(wid * 16, 16)])


@functools.partial(jax.jit, donate_argnums=())
def kernel(x, nodal_values):
    a_t, b_t, c_t, d_t = _spline_tables(nodal_values)
    x2 = x.reshape(_PLANES, _ROWS, _ROWS)

    cp = pltpu.CompilerParams(use_tc_tiling_on_sc=True)
    if "needs_layout_passes" in pltpu.CompilerParams.__dataclass_fields__:
        cp = dataclasses.replace(cp, needs_layout_passes=False)

    mesh = plsc.VectorSubcoreMesh(core_axis_name="core",
                                  subcore_axis_name="subcore")
    partial = pl.kernel(
        _sc_body,
        out_type=jax.ShapeDtypeStruct((_NW * 16,), jnp.float32),
        mesh=mesh,
        scratch_types=[
            pltpu.VMEM((_ROWS, _ROWS), jnp.float32),
            pltpu.VMEM((_ROWS, _ROWS), jnp.float32),
            pltpu.VMEM((_TAB,), jnp.float32),
            pltpu.VMEM((_TAB,), jnp.float32),
            pltpu.VMEM((_TAB,), jnp.float32),
            pltpu.VMEM((_TAB,), jnp.float32),
            pltpu.VMEM((16,), jnp.float32),
            pltpu.SemaphoreType.DMA,
            pltpu.SemaphoreType.DMA,
        ],
        compiler_params=cp,
    )(x2, a_t, b_t, c_t, d_t)
    return jnp.sum(partial)
